# baseline (speedup 1.0000x reference)
import functools

import jax
import jax.numpy as jnp
from jax import lax
from jax.experimental import pallas as pl
from jax.experimental.pallas import tpu as pltpu
from jax.experimental.pallas import tpu_sc as plsc

NC = 2
NS = 16
NW = NC * NS
L = 16

B = 16384
EMB = 64
HIST = 200
BPW = B // NW
NCHUNK = BPW // 128


def _mv_body(x_ref, w_ref, o_ref, *, col):
    o_ref[...] = jnp.sum(x_ref[...] * w_ref[:, col:col + 1], axis=0)


def _matvec(xt, w3, col, blk):
    n = xt.shape[1]
    grid = (n + blk - 1) // blk
    return pl.pallas_call(
        functools.partial(_mv_body, col=col),
        grid=(grid,),
        in_specs=[
            pl.BlockSpec((EMB, blk), lambda i: (0, i)),
            pl.BlockSpec((EMB, 3), lambda i: (0, 0)),
        ],
        out_specs=pl.BlockSpec((blk,), lambda i: (i,)),
        out_shape=jax.ShapeDtypeStruct((n,), jnp.float32),
    )(xt, w3)


def _sc_body(gidx_hbm, aidx_hbm, sidx_hbm, ss_hbm, gs_hbm, as_hbm, b16_hbm,
             out_hbm, sidx_v, cidx_v, sval_v, gval_v, aval_v, bv, outv,
             sem_s, sem_c):
    c = lax.axis_index("c")
    s = lax.axis_index("s")
    wid = s * NC + c
    base = wid * BPW

    pltpu.sync_copy(sidx_hbm.at[pl.ds(base, BPW)], sidx_v)
    song_cps = [
        pltpu.async_copy(ss_hbm.at[sidx_v.at[pl.ds(j * 128, 128)]],
                         sval_v.at[pl.ds(j * 128, 128)], sem_s)
        for j in range(NCHUNK)
    ]

    pltpu.sync_copy(gidx_hbm, cidx_v.at[0, pl.ds(0, HIST)])
    pltpu.sync_copy(aidx_hbm, cidx_v.at[1, pl.ds(0, HIST)])

    zeros = jnp.zeros((L,), jnp.float32)
    gval_v[pl.ds(192, L)] = zeros
    aval_v[pl.ds(192, L)] = zeros

    const_cps = [
        pltpu.async_copy(gs_hbm.at[cidx_v.at[0, pl.ds(0, 128)]],
                         gval_v.at[pl.ds(0, 128)], sem_c),
        pltpu.async_copy(gs_hbm.at[cidx_v.at[0, pl.ds(128, 72)]],
                         gval_v.at[pl.ds(128, 72)], sem_c),
        pltpu.async_copy(as_hbm.at[cidx_v.at[1, pl.ds(0, 128)]],
                         aval_v.at[pl.ds(0, 128)], sem_c),
        pltpu.async_copy(as_hbm.at[cidx_v.at[1, pl.ds(128, 72)]],
                         aval_v.at[pl.ds(128, 72)], sem_c),
    ]
    pltpu.sync_copy(b16_hbm, bv)

    lane = lax.iota(jnp.int32, L)
    dnums = lax.GatherDimensionNumbers(
        offset_dims=(), collapsed_slice_dims=(0,), start_index_map=(0,))

    def allsum(v):
        for step in (1, 2, 4, 8):
            p = lax.gather(v, (lane ^ step)[:, None], dnums, slice_sizes=(1,),
                           mode=lax.GatherScatterMode.PROMISE_IN_BOUNDS)
            v = v + p
        return v

    for cp in const_cps:
        cp.wait()

    gtot = zeros
    atot = zeros
    for t in range(13):
        gtot = gtot + gval_v[pl.ds(t * L, L)]
        atot = atot + aval_v[pl.ds(t * L, L)]
    cconst = (allsum(gtot) + allsum(atot)) * (1.0 / HIST) + allsum(bv[...])

    for cp in song_cps:
        cp.wait()

    def group(g, _):
        outv[pl.ds(g * L, L)] = sval_v[pl.ds(g * L, L)] + cconst
        return 0

    lax.fori_loop(0, BPW // L, group, 0)

    pltpu.sync_copy(outv, out_hbm.at[pl.ds(base, BPW)])


@jax.jit
def _run(gidx, aidx, sidx, song_scores, genre_scores, artist_scores, b16):
    mesh = plsc.VectorSubcoreMesh(core_axis_name="c", subcore_axis_name="s",
                                  num_cores=NC, num_subcores=NS)
    return pl.kernel(
        _sc_body,
        out_type=jax.ShapeDtypeStruct((B,), jnp.float32),
        mesh=mesh,
        scratch_types=[
            pltpu.VMEM((BPW,), jnp.int32),
            pltpu.VMEM((2, HIST), jnp.int32),
            pltpu.VMEM((BPW,), jnp.float32),
            pltpu.VMEM((208,), jnp.float32),
            pltpu.VMEM((208,), jnp.float32),
            pltpu.VMEM((L,), jnp.float32),
            pltpu.VMEM((BPW,), jnp.float32),
            pltpu.SemaphoreType.DMA,
            pltpu.SemaphoreType.DMA,
        ],
        compiler_params=pltpu.CompilerParams(needs_layout_passes=False),
    )(gidx, aidx, sidx, song_scores, genre_scores, artist_scores, b16)


def kernel(genre_indices, artist_indices, song_indices, song_table,
           genre_table, artist_table, fc_w, fc_b):
    w3 = fc_w.reshape(3, EMB).T
    song_scores = _matvec(song_table.T, w3, 2, 16384)
    genre_scores = _matvec(genre_table.T, w3, 0, 1024)
    artist_scores = _matvec(artist_table.T, w3, 1, 8192)
    b16 = jnp.pad(fc_b.reshape(-1), (0, L - 1))
    return _run(genre_indices.astype(jnp.int32),
                artist_indices.astype(jnp.int32),
                song_indices.astype(jnp.int32),
                song_scores, genre_scores, artist_scores, b16)


# speedup vs baseline: 1.1057x; 1.1057x over previous
import functools

import jax
import jax.numpy as jnp
from jax import lax
from jax.experimental import pallas as pl
from jax.experimental.pallas import tpu as pltpu
from jax.experimental.pallas import tpu_sc as plsc

NC = 2
NS = 16
NW = NC * NS
L = 16

B = 16384
EMB = 64
HIST = 200
BPW = B // NW
NCHUNK = BPW // 128


def _mv_body(x_ref, w_ref, o_ref, *, col):
    o_ref[...] = jnp.sum(x_ref[...] * w_ref[:, col:col + 1], axis=0)


def _matvec(xt, w3, col, blk):
    n = xt.shape[1]
    grid = (n + blk - 1) // blk
    return pl.pallas_call(
        functools.partial(_mv_body, col=col),
        grid=(grid,),
        in_specs=[
            pl.BlockSpec((EMB, blk), lambda i: (0, i)),
            pl.BlockSpec((EMB, 3), lambda i: (0, 0)),
        ],
        out_specs=pl.BlockSpec((blk,), lambda i: (i,)),
        out_shape=jax.ShapeDtypeStruct((n,), jnp.float32),
    )(xt, w3)


def _sc_body(gidx_hbm, aidx_hbm, sidx_hbm, ss_hbm, gs_hbm, as_hbm, b16_hbm,
             out_hbm, sidx_v, cidx_v, sval_v, gval_v, aval_v, bv, outv,
             sem_s, sem_c):
    c = lax.axis_index("c")
    s = lax.axis_index("s")
    wid = s * NC + c
    base = wid * BPW

    pltpu.sync_copy(sidx_hbm.at[pl.ds(base, BPW)], sidx_v)
    song_cps = [
        pltpu.async_copy(ss_hbm.at[sidx_v.at[pl.ds(j * 128, 128)]],
                         sval_v.at[pl.ds(j * 128, 128)], sem_s)
        for j in range(NCHUNK)
    ]

    pltpu.sync_copy(gidx_hbm, cidx_v.at[0, pl.ds(0, HIST)])
    pltpu.sync_copy(aidx_hbm, cidx_v.at[1, pl.ds(0, HIST)])

    zeros = jnp.zeros((L,), jnp.float32)
    gval_v[pl.ds(192, L)] = zeros
    aval_v[pl.ds(192, L)] = zeros

    const_cps = [
        pltpu.async_copy(gs_hbm.at[cidx_v.at[0, pl.ds(0, 128)]],
                         gval_v.at[pl.ds(0, 128)], sem_c),
        pltpu.async_copy(gs_hbm.at[cidx_v.at[0, pl.ds(128, 72)]],
                         gval_v.at[pl.ds(128, 72)], sem_c),
        pltpu.async_copy(as_hbm.at[cidx_v.at[1, pl.ds(0, 128)]],
                         aval_v.at[pl.ds(0, 128)], sem_c),
        pltpu.async_copy(as_hbm.at[cidx_v.at[1, pl.ds(128, 72)]],
                         aval_v.at[pl.ds(128, 72)], sem_c),
    ]
    pltpu.sync_copy(b16_hbm, bv)

    lane = lax.iota(jnp.int32, L)
    dnums = lax.GatherDimensionNumbers(
        offset_dims=(), collapsed_slice_dims=(0,), start_index_map=(0,))

    def allsum(v):
        for step in (1, 2, 4, 8):
            p = lax.gather(v, (lane ^ step)[:, None], dnums, slice_sizes=(1,),
                           mode=lax.GatherScatterMode.PROMISE_IN_BOUNDS)
            v = v + p
        return v

    for cp in const_cps:
        cp.wait()

    gtot = zeros
    atot = zeros
    for t in range(13):
        gtot = gtot + gval_v[pl.ds(t * L, L)]
        atot = atot + aval_v[pl.ds(t * L, L)]
    cconst = (allsum(gtot) + allsum(atot)) * (1.0 / HIST) + allsum(bv[...])

    for cp in song_cps:
        cp.wait()

    def group(g, _):
        outv[pl.ds(g * L, L)] = sval_v[pl.ds(g * L, L)] + cconst
        return 0

    lax.fori_loop(0, BPW // L, group, 0)

    pltpu.sync_copy(outv, out_hbm.at[pl.ds(base, BPW)])


@jax.jit
def _run(gidx, aidx, sidx, song_scores, genre_scores, artist_scores, b16):
    mesh = plsc.VectorSubcoreMesh(core_axis_name="c", subcore_axis_name="s",
                                  num_cores=NC, num_subcores=NS)
    return pl.kernel(
        _sc_body,
        out_type=jax.ShapeDtypeStruct((B,), jnp.float32),
        mesh=mesh,
        scratch_types=[
            pltpu.VMEM((BPW,), jnp.int32),
            pltpu.VMEM((2, HIST), jnp.int32),
            pltpu.VMEM((BPW,), jnp.float32),
            pltpu.VMEM((208,), jnp.float32),
            pltpu.VMEM((208,), jnp.float32),
            pltpu.VMEM((L,), jnp.float32),
            pltpu.VMEM((BPW,), jnp.float32),
            pltpu.SemaphoreType.DMA,
            pltpu.SemaphoreType.DMA,
        ],
        compiler_params=pltpu.CompilerParams(needs_layout_passes=False),
    )(gidx, aidx, sidx, song_scores, genre_scores, artist_scores, b16)


def kernel(genre_indices, artist_indices, song_indices, song_table,
           genre_table, artist_table, fc_w, fc_b):
    w3 = fc_w.reshape(3, EMB).T
    song_scores = _matvec(song_table.T, w3, 2, 32768)
    genre_scores = _matvec(genre_table.T, w3, 0, 1024)
    artist_scores = _matvec(artist_table.T, w3, 1, 8192)
    b16 = jnp.pad(fc_b.reshape(-1), (0, L - 1))
    return _run(genre_indices.astype(jnp.int32),
                artist_indices.astype(jnp.int32),
                song_indices.astype(jnp.int32),
                song_scores, genre_scores, artist_scores, b16)


# speedup vs baseline: 1.1196x; 1.0126x over previous
import functools

import jax
import jax.numpy as jnp
from jax import lax
from jax.experimental import pallas as pl
from jax.experimental.pallas import tpu as pltpu
from jax.experimental.pallas import tpu_sc as plsc

NC = 2
NS = 16
NW = NC * NS
L = 16

B = 16384
EMB = 64
HIST = 200
BPW = B // NW
NCHUNK = BPW // 128


def _mv_body(x_ref, w_ref, o_ref, *, col):
    o_ref[...] = jnp.sum(x_ref[...] * w_ref[:, col:col + 1], axis=0)


def _matvec(xt, w3, col, blk):
    n = xt.shape[1]
    grid = (n + blk - 1) // blk
    return pl.pallas_call(
        functools.partial(_mv_body, col=col),
        grid=(grid,),
        in_specs=[
            pl.BlockSpec((EMB, blk), lambda i: (0, i)),
            pl.BlockSpec((EMB, 3), lambda i: (0, 0)),
        ],
        out_specs=pl.BlockSpec((blk,), lambda i: (i,)),
        out_shape=jax.ShapeDtypeStruct((n,), jnp.float32),
    )(xt, w3)


def _sc_body(gidx_hbm, aidx_hbm, sidx_hbm, ss_hbm, gs_hbm, as_hbm, b16_hbm,
             out_hbm, sidx_v, cidx_v, sval_v, gval_v, aval_v, bv, outv,
             sem_s, sem_c):
    c = lax.axis_index("c")
    s = lax.axis_index("s")
    wid = s * NC + c
    base = wid * BPW

    pltpu.sync_copy(sidx_hbm.at[pl.ds(base, BPW)], sidx_v)
    song_cps = [
        pltpu.async_copy(ss_hbm.at[sidx_v.at[pl.ds(j * 128, 128)]],
                         sval_v.at[pl.ds(j * 128, 128)], sem_s)
        for j in range(NCHUNK)
    ]

    pltpu.sync_copy(gidx_hbm, cidx_v.at[0, pl.ds(0, HIST)])
    pltpu.sync_copy(aidx_hbm, cidx_v.at[1, pl.ds(0, HIST)])

    zeros = jnp.zeros((L,), jnp.float32)
    gval_v[pl.ds(192, L)] = zeros
    aval_v[pl.ds(192, L)] = zeros

    const_cps = [
        pltpu.async_copy(gs_hbm.at[cidx_v.at[0, pl.ds(0, 128)]],
                         gval_v.at[pl.ds(0, 128)], sem_c),
        pltpu.async_copy(gs_hbm.at[cidx_v.at[0, pl.ds(128, 72)]],
                         gval_v.at[pl.ds(128, 72)], sem_c),
        pltpu.async_copy(as_hbm.at[cidx_v.at[1, pl.ds(0, 128)]],
                         aval_v.at[pl.ds(0, 128)], sem_c),
        pltpu.async_copy(as_hbm.at[cidx_v.at[1, pl.ds(128, 72)]],
                         aval_v.at[pl.ds(128, 72)], sem_c),
    ]
    pltpu.sync_copy(b16_hbm, bv)

    lane = lax.iota(jnp.int32, L)
    dnums = lax.GatherDimensionNumbers(
        offset_dims=(), collapsed_slice_dims=(0,), start_index_map=(0,))

    def allsum(v):
        for step in (1, 2, 4, 8):
            p = lax.gather(v, (lane ^ step)[:, None], dnums, slice_sizes=(1,),
                           mode=lax.GatherScatterMode.PROMISE_IN_BOUNDS)
            v = v + p
        return v

    for cp in const_cps:
        cp.wait()

    gtot = zeros
    atot = zeros
    for t in range(13):
        gtot = gtot + gval_v[pl.ds(t * L, L)]
        atot = atot + aval_v[pl.ds(t * L, L)]
    cconst = (allsum(gtot) + allsum(atot)) * (1.0 / HIST) + allsum(bv[...])

    for cp in song_cps:
        cp.wait()

    def group(g, _):
        outv[pl.ds(g * L, L)] = sval_v[pl.ds(g * L, L)] + cconst
        return 0

    lax.fori_loop(0, BPW // L, group, 0)

    pltpu.sync_copy(outv, out_hbm.at[pl.ds(base, BPW)])


@jax.jit
def _run(gidx, aidx, sidx, song_scores, genre_scores, artist_scores, b16):
    mesh = plsc.VectorSubcoreMesh(core_axis_name="c", subcore_axis_name="s",
                                  num_cores=NC, num_subcores=NS)
    return pl.kernel(
        _sc_body,
        out_type=jax.ShapeDtypeStruct((B,), jnp.float32),
        mesh=mesh,
        scratch_types=[
            pltpu.VMEM((BPW,), jnp.int32),
            pltpu.VMEM((2, HIST), jnp.int32),
            pltpu.VMEM((BPW,), jnp.float32),
            pltpu.VMEM((208,), jnp.float32),
            pltpu.VMEM((208,), jnp.float32),
            pltpu.VMEM((L,), jnp.float32),
            pltpu.VMEM((BPW,), jnp.float32),
            pltpu.SemaphoreType.DMA,
            pltpu.SemaphoreType.DMA,
        ],
        compiler_params=pltpu.CompilerParams(needs_layout_passes=False),
    )(gidx, aidx, sidx, song_scores, genre_scores, artist_scores, b16)


def kernel(genre_indices, artist_indices, song_indices, song_table,
           genre_table, artist_table, fc_w, fc_b):
    w3 = fc_w.reshape(3, EMB).T
    song_scores = _matvec(song_table.T, w3, 2, 65536)
    genre_scores = _matvec(genre_table.T, w3, 0, 1024)
    artist_scores = _matvec(artist_table.T, w3, 1, 16384)
    b16 = jnp.pad(fc_b.reshape(-1), (0, L - 1))
    return _run(genre_indices.astype(jnp.int32),
                artist_indices.astype(jnp.int32),
                song_indices.astype(jnp.int32),
                song_scores, genre_scores, artist_scores, b16)


# speedup vs baseline: 1.1751x; 1.0495x over previous
import functools

import jax
import jax.numpy as jnp
from jax import lax
from jax.experimental import pallas as pl
from jax.experimental.pallas import tpu as pltpu
from jax.experimental.pallas import tpu_sc as plsc

NC = 2
NS = 16
NW = NC * NS
L = 16

B = 16384
EMB = 64
HIST = 200
BPW = B // NW
NCHUNK = BPW // 128


SBLK = 65536
ABLK = 16384
NSONG = 1000000
NART = 100000
NGEN = 1000
SGRID = (NSONG + SBLK - 1) // SBLK
AGRID = (NART + ABLK - 1) // ABLK


def _mv3_body(song_ref, art_ref, gen_ref, w_ref, so_ref, ao_ref, go_ref):
    so_ref[...] = jnp.sum(song_ref[...] * w_ref[:, 2:3], axis=0)
    ao_ref[...] = jnp.sum(art_ref[...] * w_ref[:, 1:2], axis=0)
    go_ref[...] = jnp.sum(gen_ref[...] * w_ref[:, 0:1], axis=0)


def _scores(songt, artistt, genret, w3):
    return pl.pallas_call(
        _mv3_body,
        grid=(SGRID,),
        in_specs=[
            pl.BlockSpec((EMB, SBLK), lambda i: (0, i)),
            pl.BlockSpec((EMB, ABLK), lambda i: (0, jnp.minimum(i, AGRID - 1))),
            pl.BlockSpec((EMB, 1024), lambda i: (0, 0)),
            pl.BlockSpec((EMB, 3), lambda i: (0, 0)),
        ],
        out_specs=[
            pl.BlockSpec((SBLK,), lambda i: (i,)),
            pl.BlockSpec((ABLK,), lambda i: (jnp.minimum(i, AGRID - 1),)),
            pl.BlockSpec((1024,), lambda i: (0,)),
        ],
        out_shape=[
            jax.ShapeDtypeStruct((NSONG,), jnp.float32),
            jax.ShapeDtypeStruct((NART,), jnp.float32),
            jax.ShapeDtypeStruct((NGEN,), jnp.float32),
        ],
    )(songt, artistt, genret, w3)


def _sc_body(gidx_hbm, aidx_hbm, sidx_hbm, ss_hbm, gs_hbm, as_hbm, b16_hbm,
             out_hbm, sidx_v, cidx_v, sval_v, gval_v, aval_v, bv, outv,
             sem_s, sem_c):
    c = lax.axis_index("c")
    s = lax.axis_index("s")
    wid = s * NC + c
    base = wid * BPW

    stage_cps = [
        pltpu.async_copy(sidx_hbm.at[pl.ds(base, BPW)], sidx_v, sem_c),
        pltpu.async_copy(gidx_hbm, cidx_v.at[0, pl.ds(0, HIST)], sem_c),
        pltpu.async_copy(aidx_hbm, cidx_v.at[1, pl.ds(0, HIST)], sem_c),
        pltpu.async_copy(b16_hbm, bv, sem_c),
    ]

    zeros = jnp.zeros((L,), jnp.float32)
    gval_v[pl.ds(192, L)] = zeros
    aval_v[pl.ds(192, L)] = zeros

    for cp in stage_cps:
        cp.wait()

    song_cps = [
        pltpu.async_copy(ss_hbm.at[sidx_v.at[pl.ds(j * 128, 128)]],
                         sval_v.at[pl.ds(j * 128, 128)], sem_s)
        for j in range(NCHUNK)
    ]
    const_cps = [
        pltpu.async_copy(gs_hbm.at[cidx_v.at[0, pl.ds(0, 128)]],
                         gval_v.at[pl.ds(0, 128)], sem_c),
        pltpu.async_copy(gs_hbm.at[cidx_v.at[0, pl.ds(128, 72)]],
                         gval_v.at[pl.ds(128, 72)], sem_c),
        pltpu.async_copy(as_hbm.at[cidx_v.at[1, pl.ds(0, 128)]],
                         aval_v.at[pl.ds(0, 128)], sem_c),
        pltpu.async_copy(as_hbm.at[cidx_v.at[1, pl.ds(128, 72)]],
                         aval_v.at[pl.ds(128, 72)], sem_c),
    ]

    lane = lax.iota(jnp.int32, L)
    dnums = lax.GatherDimensionNumbers(
        offset_dims=(), collapsed_slice_dims=(0,), start_index_map=(0,))

    def allsum(v):
        for step in (1, 2, 4, 8):
            p = lax.gather(v, (lane ^ step)[:, None], dnums, slice_sizes=(1,),
                           mode=lax.GatherScatterMode.PROMISE_IN_BOUNDS)
            v = v + p
        return v

    for cp in const_cps:
        cp.wait()

    gtot = zeros
    atot = zeros
    for t in range(13):
        gtot = gtot + gval_v[pl.ds(t * L, L)]
        atot = atot + aval_v[pl.ds(t * L, L)]
    cconst = (allsum(gtot) + allsum(atot)) * (1.0 / HIST) + allsum(bv[...])

    for cp in song_cps:
        cp.wait()

    def group(g, _):
        outv[pl.ds(g * L, L)] = sval_v[pl.ds(g * L, L)] + cconst
        return 0

    lax.fori_loop(0, BPW // L, group, 0)

    pltpu.sync_copy(outv, out_hbm.at[pl.ds(base, BPW)])


@jax.jit
def _run(gidx, aidx, sidx, song_scores, genre_scores, artist_scores, b16):
    mesh = plsc.VectorSubcoreMesh(core_axis_name="c", subcore_axis_name="s",
                                  num_cores=NC, num_subcores=NS)
    return pl.kernel(
        _sc_body,
        out_type=jax.ShapeDtypeStruct((B,), jnp.float32),
        mesh=mesh,
        scratch_types=[
            pltpu.VMEM((BPW,), jnp.int32),
            pltpu.VMEM((2, HIST), jnp.int32),
            pltpu.VMEM((BPW,), jnp.float32),
            pltpu.VMEM((208,), jnp.float32),
            pltpu.VMEM((208,), jnp.float32),
            pltpu.VMEM((L,), jnp.float32),
            pltpu.VMEM((BPW,), jnp.float32),
            pltpu.SemaphoreType.DMA,
            pltpu.SemaphoreType.DMA,
        ],
        compiler_params=pltpu.CompilerParams(needs_layout_passes=False),
    )(gidx, aidx, sidx, song_scores, genre_scores, artist_scores, b16)


def kernel(genre_indices, artist_indices, song_indices, song_table,
           genre_table, artist_table, fc_w, fc_b):
    w3 = fc_w.reshape(3, EMB).T
    song_scores, artist_scores, genre_scores = _scores(
        song_table.T, artist_table.T, genre_table.T, w3)
    b16 = jnp.pad(fc_b.reshape(-1), (0, L - 1))
    return _run(genre_indices.astype(jnp.int32),
                artist_indices.astype(jnp.int32),
                song_indices.astype(jnp.int32),
                song_scores, genre_scores, artist_scores, b16)


# speedup vs baseline: 1.2034x; 1.0241x over previous
import functools

import jax
import jax.numpy as jnp
from jax import lax
from jax.experimental import pallas as pl
from jax.experimental.pallas import tpu as pltpu
from jax.experimental.pallas import tpu_sc as plsc

NC = 2
NS = 16
NW = NC * NS
L = 16

B = 16384
EMB = 64
HIST = 200
BPW = B // NW
NCHUNK = BPW // 128


SBLK = 65536
ABLK = 16384
NSONG = 1000000
NART = 100000
NGEN = 1000
SGRID = (NSONG + SBLK - 1) // SBLK
AGRID = (NART + ABLK - 1) // ABLK


def _dotrow(w_row, x):
    r = lax.dot_general(w_row, x, (((1,), (0,)), ((), ())),
                        preferred_element_type=jnp.float32)
    return r[0]


def _mv3_body(song_ref, art_ref, gen_ref, w_ref, so_ref, ao_ref, go_ref):
    so_ref[...] = _dotrow(w_ref[2:3, :], song_ref[...])
    ao_ref[...] = _dotrow(w_ref[1:2, :], art_ref[...])
    go_ref[...] = _dotrow(w_ref[0:1, :], gen_ref[...])


def _scores(songt, artistt, genret, w3):
    return pl.pallas_call(
        _mv3_body,
        grid=(SGRID,),
        in_specs=[
            pl.BlockSpec((EMB, SBLK), lambda i: (0, i)),
            pl.BlockSpec((EMB, ABLK), lambda i: (0, jnp.minimum(i, AGRID - 1))),
            pl.BlockSpec((EMB, 1024), lambda i: (0, 0)),
            pl.BlockSpec((3, EMB), lambda i: (0, 0)),
        ],
        out_specs=[
            pl.BlockSpec((SBLK,), lambda i: (i,)),
            pl.BlockSpec((ABLK,), lambda i: (jnp.minimum(i, AGRID - 1),)),
            pl.BlockSpec((1024,), lambda i: (0,)),
        ],
        out_shape=[
            jax.ShapeDtypeStruct((NSONG,), jnp.float32),
            jax.ShapeDtypeStruct((NART,), jnp.float32),
            jax.ShapeDtypeStruct((NGEN,), jnp.float32),
        ],
    )(songt, artistt, genret, w3)


def _sc_body(gidx_hbm, aidx_hbm, sidx_hbm, ss_hbm, gs_hbm, as_hbm, b16_hbm,
             out_hbm, sidx_v, cidx_v, sval_v, gval_v, aval_v, bv, outv,
             sem_s, sem_c):
    c = lax.axis_index("c")
    s = lax.axis_index("s")
    wid = s * NC + c
    base = wid * BPW

    stage_cps = [
        pltpu.async_copy(sidx_hbm.at[pl.ds(base, BPW)], sidx_v, sem_c),
        pltpu.async_copy(gidx_hbm, cidx_v.at[0, pl.ds(0, HIST)], sem_c),
        pltpu.async_copy(aidx_hbm, cidx_v.at[1, pl.ds(0, HIST)], sem_c),
        pltpu.async_copy(b16_hbm, bv, sem_c),
    ]

    zeros = jnp.zeros((L,), jnp.float32)
    gval_v[pl.ds(192, L)] = zeros
    aval_v[pl.ds(192, L)] = zeros

    for cp in stage_cps:
        cp.wait()

    song_cps = [
        pltpu.async_copy(ss_hbm.at[sidx_v.at[pl.ds(j * 128, 128)]],
                         sval_v.at[pl.ds(j * 128, 128)], sem_s)
        for j in range(NCHUNK)
    ]
    const_cps = [
        pltpu.async_copy(gs_hbm.at[cidx_v.at[0, pl.ds(0, 128)]],
                         gval_v.at[pl.ds(0, 128)], sem_c),
        pltpu.async_copy(gs_hbm.at[cidx_v.at[0, pl.ds(128, 72)]],
                         gval_v.at[pl.ds(128, 72)], sem_c),
        pltpu.async_copy(as_hbm.at[cidx_v.at[1, pl.ds(0, 128)]],
                         aval_v.at[pl.ds(0, 128)], sem_c),
        pltpu.async_copy(as_hbm.at[cidx_v.at[1, pl.ds(128, 72)]],
                         aval_v.at[pl.ds(128, 72)], sem_c),
    ]

    lane = lax.iota(jnp.int32, L)
    dnums = lax.GatherDimensionNumbers(
        offset_dims=(), collapsed_slice_dims=(0,), start_index_map=(0,))

    def allsum(v):
        for step in (1, 2, 4, 8):
            p = lax.gather(v, (lane ^ step)[:, None], dnums, slice_sizes=(1,),
                           mode=lax.GatherScatterMode.PROMISE_IN_BOUNDS)
            v = v + p
        return v

    for cp in const_cps:
        cp.wait()

    gtot = zeros
    atot = zeros
    for t in range(13):
        gtot = gtot + gval_v[pl.ds(t * L, L)]
        atot = atot + aval_v[pl.ds(t * L, L)]
    cconst = (allsum(gtot) + allsum(atot)) * (1.0 / HIST) + allsum(bv[...])

    for cp in song_cps:
        cp.wait()

    def group(g, _):
        outv[pl.ds(g * L, L)] = sval_v[pl.ds(g * L, L)] + cconst
        return 0

    lax.fori_loop(0, BPW // L, group, 0)

    pltpu.sync_copy(outv, out_hbm.at[pl.ds(base, BPW)])


@jax.jit
def _run(gidx, aidx, sidx, song_scores, genre_scores, artist_scores, b16):
    mesh = plsc.VectorSubcoreMesh(core_axis_name="c", subcore_axis_name="s",
                                  num_cores=NC, num_subcores=NS)
    return pl.kernel(
        _sc_body,
        out_type=jax.ShapeDtypeStruct((B,), jnp.float32),
        mesh=mesh,
        scratch_types=[
            pltpu.VMEM((BPW,), jnp.int32),
            pltpu.VMEM((2, HIST), jnp.int32),
            pltpu.VMEM((BPW,), jnp.float32),
            pltpu.VMEM((208,), jnp.float32),
            pltpu.VMEM((208,), jnp.float32),
            pltpu.VMEM((L,), jnp.float32),
            pltpu.VMEM((BPW,), jnp.float32),
            pltpu.SemaphoreType.DMA,
            pltpu.SemaphoreType.DMA,
        ],
        compiler_params=pltpu.CompilerParams(needs_layout_passes=False),
    )(gidx, aidx, sidx, song_scores, genre_scores, artist_scores, b16)


def kernel(genre_indices, artist_indices, song_indices, song_table,
           genre_table, artist_table, fc_w, fc_b):
    w3 = fc_w.reshape(3, EMB)
    song_scores, artist_scores, genre_scores = _scores(
        song_table.T, artist_table.T, genre_table.T, w3)
    b16 = jnp.pad(fc_b.reshape(-1), (0, L - 1))
    return _run(genre_indices.astype(jnp.int32),
                artist_indices.astype(jnp.int32),
                song_indices.astype(jnp.int32),
                song_scores, genre_scores, artist_scores, b16)
